# baseline (speedup 1.0000x reference)
import functools

import jax
import jax.numpy as jnp
from jax import lax
from jax.experimental import pallas as pl
from jax.experimental.pallas import tpu as pltpu
from jax.experimental.pallas import tpu_sc as plsc

_EMB_DIM = 4096
_CLAP_DIM = 768

_NC, _NS = 2, 16
_NW = _NC * _NS
_CHUNK = 16


def _sc_gather(emb, ids):
    n = ids.shape[0]
    width = emb.shape[1]
    per_tile = n // _NW
    mesh = plsc.VectorSubcoreMesh(core_axis_name="c", subcore_axis_name="s")

    @functools.partial(
        pl.kernel,
        out_type=jax.ShapeDtypeStruct((n, width), emb.dtype),
        mesh=mesh,
        scratch_types=[
            pltpu.VMEM((_CHUNK,), jnp.int32),
            pltpu.VMEM((_CHUNK, width), emb.dtype),
            pltpu.SemaphoreType.DMA,
        ],
    )
    def gather_kernel(emb_hbm, ids_hbm, out_hbm, idx_v, rows_v, sem):
        wid = lax.axis_index("s") * _NC + lax.axis_index("c")
        base = wid * per_tile

        @pl.loop(0, per_tile, step=_CHUNK)
        def _(off):
            pltpu.sync_copy(ids_hbm.at[pl.ds(base + off, _CHUNK)], idx_v)
            pltpu.async_copy(emb_hbm.at[idx_v], rows_v, sem).wait()
            pltpu.sync_copy(rows_v, out_hbm.at[pl.ds(base + off, _CHUNK)])

    return gather_kernel(emb, ids)


_DECODE_MASK = 0x8FFFE000 - 0x100000000
_DECODE_SCALE = 2.0**112


def _tc_project_merge(clap, w, gathered_bf16, pos):
    n = clap.shape[0]
    blk = 512
    grid = (n // blk,)

    def body(clap_ref, w_ref, gath_ref, pos_ref, out_ref):
        audio = lax.dot_general(
            clap_ref[...], w_ref[...],
            (((1,), (1,)), ((), ())),
            preferred_element_type=jnp.float32,
        )
        bits = lax.bitcast_convert_type(
            gath_ref[...].astype(jnp.float32), jnp.int32
        )
        gf = lax.bitcast_convert_type(
            lax.shift_right_arithmetic(bits, 3) & jnp.int32(_DECODE_MASK),
            jnp.float32,
        ) * jnp.float32(_DECODE_SCALE)
        mask = pos_ref[...] > 0
        out_ref[...] = jnp.where(mask, audio, gf)

    return pl.pallas_call(
        body,
        grid=grid,
        in_specs=[
            pl.BlockSpec((blk, _CLAP_DIM), lambda i: (i, 0)),
            pl.BlockSpec((_EMB_DIM, _CLAP_DIM), lambda i: (0, 0)),
            pl.BlockSpec((blk, _EMB_DIM), lambda i: (i, 0)),
            pl.BlockSpec((blk, 1), lambda i: (i, 0)),
        ],
        out_specs=pl.BlockSpec((blk, _EMB_DIM), lambda i: (i, 0)),
        out_shape=jax.ShapeDtypeStruct((n, _EMB_DIM), jnp.float32),
    )(clap, w, gathered_bf16, pos)


def kernel(input_ids, clap_rep, pos_id, emb, W):
    b, s = input_ids.shape
    n = b * s
    ids = input_ids.reshape(n)
    vocab = emb.shape[0]
    emb_i32 = lax.bitcast_convert_type(
        emb.reshape(vocab, _EMB_DIM // 2, 2), jnp.int32
    )
    gathered_i32 = _sc_gather(emb_i32, ids)
    gathered_bf16 = lax.bitcast_convert_type(gathered_i32, jnp.bfloat16).reshape(
        n, _EMB_DIM
    )
    out = _tc_project_merge(
        clap_rep.reshape(n, _CLAP_DIM), W, gathered_bf16, pos_id.reshape(n, 1)
    )
    return out.reshape(b, s, _EMB_DIM)


# speedup vs baseline: 16.7369x; 16.7369x over previous
import functools

import jax
import jax.numpy as jnp
from jax import lax
from jax.experimental import pallas as pl
from jax.experimental.pallas import tpu as pltpu
from jax.experimental.pallas import tpu_sc as plsc

_EMB_DIM = 4096
_CLAP_DIM = 768

_NC, _NS = 2, 16
_NW = _NC * _NS
_CHUNK = 8


def _sc_gather(emb, ids2):
    n = ids2.shape[0] * ids2.shape[1]
    per_tile = n // _NW
    n_chunks = per_tile // _CHUNK
    v_even = emb.shape[0] - 1
    mesh = plsc.VectorSubcoreMesh(core_axis_name="c", subcore_axis_name="s")

    @functools.partial(
        pl.kernel,
        out_type=jax.ShapeDtypeStruct((n, _EMB_DIM), jnp.int32),
        mesh=mesh,
        scratch_types=[
            pltpu.VMEM((n_chunks, _CHUNK), jnp.int32),
            pltpu.VMEM((_CHUNK, _EMB_DIM), jnp.int32),
            pltpu.VMEM((_CHUNK, _EMB_DIM), jnp.int32),
            pltpu.SemaphoreType.DMA,
            pltpu.SemaphoreType.DMA,
            pltpu.SemaphoreType.DMA,
            pltpu.SemaphoreType.DMA,
        ],
    )
    def gather_kernel(emb_hbm, ids_hbm, out_hbm, idx_v, buf0, buf1, gs0, gs1,
                      ws0, ws1):
        emb_i32 = emb_hbm.at[pl.ds(0, v_even)].bitcast(jnp.int32)
        wid = lax.axis_index("s") * _NC + lax.axis_index("c")
        base = wid * per_tile

        pltpu.sync_copy(ids_hbm.at[pl.ds(wid * n_chunks, n_chunks)], idx_v)

        def gather(c, buf, sem):
            pltpu.async_copy(emb_i32.at[idx_v.at[c]], buf, sem)

        def gather_wait(c, buf, sem):
            pltpu.make_async_copy(emb_i32.at[idx_v.at[c]], buf, sem).wait()

        def _out_slice(c):
            off = pl.multiple_of(base + c * _CHUNK, _CHUNK)
            return out_hbm.at[pl.ds(off, _CHUNK)]

        def wback(c, buf, sem):
            pltpu.async_copy(buf, _out_slice(c), sem)

        def wback_wait(c, buf, sem):
            pltpu.make_async_copy(buf, _out_slice(c), sem).wait()

        gather(0, buf0, gs0)
        gather(1, buf1, gs1)

        @pl.loop(0, n_chunks // 2)
        def _(i):
            c0 = 2 * i
            gather_wait(c0, buf0, gs0)
            wback(c0, buf0, ws0)
            gather_wait(c0 + 1, buf1, gs1)
            wback(c0 + 1, buf1, ws1)

            @pl.when(i < n_chunks // 2 - 1)
            def _():
                wback_wait(c0, buf0, ws0)
                gather(c0 + 2, buf0, gs0)
                wback_wait(c0 + 1, buf1, ws1)
                gather(c0 + 3, buf1, gs1)

        wback_wait(n_chunks - 2, buf0, ws0)
        wback_wait(n_chunks - 1, buf1, ws1)

    return gather_kernel(emb, ids2)


def _tc_project_merge(clap, w, gathered, sel):
    n = clap.shape[0]
    blk = 512
    grid = (n // blk,)

    def body(clap_ref, w_ref, gath_ref, sel_ref, out_ref):
        audio = lax.dot_general(
            clap_ref[...], w_ref[...],
            (((1,), (1,)), ((), ())),
            preferred_element_type=jnp.float32,
        )
        sel = sel_ref[...]
        words = gath_ref[...]
        h = jnp.where(sel > 0,
                      lax.shift_right_logical(words, 16),
                      words & jnp.int32(0xFFFF))
        f32bits = (lax.shift_left(h & jnp.int32(0x8000), 16)
                   | lax.shift_left(h & jnp.int32(0x7FFF), 13))
        gf = lax.bitcast_convert_type(f32bits, jnp.float32) * jnp.float32(
            2.0**112)
        out_ref[...] = jnp.where(sel < 0, audio, gf)

    return pl.pallas_call(
        body,
        grid=grid,
        in_specs=[
            pl.BlockSpec((blk, _CLAP_DIM), lambda i: (i, 0)),
            pl.BlockSpec((_EMB_DIM, _CLAP_DIM), lambda i: (0, 0)),
            pl.BlockSpec((blk, _EMB_DIM), lambda i: (i, 0)),
            pl.BlockSpec((blk, 1), lambda i: (i, 0)),
        ],
        out_specs=pl.BlockSpec((blk, _EMB_DIM), lambda i: (i, 0)),
        out_shape=jax.ShapeDtypeStruct((n, _EMB_DIM), jnp.float32),
    )(clap, w, gathered, sel)


def kernel(input_ids, clap_rep, pos_id, emb, W):
    b, s = input_ids.shape
    n = b * s
    per_tile = n // _NW
    n_chunks = per_tile // _CHUNK
    a_content = emb.shape[0] - 1
    ids = input_ids.reshape(n)
    audio = pos_id.reshape(n) > 0
    word_ids = jnp.where(audio, 0, ids) >> 1
    sel = jnp.where(audio, -1, ids & 1).astype(jnp.int32)
    gathered = _sc_gather(emb, word_ids.reshape(_NW * n_chunks, _CHUNK))
    out = _tc_project_merge(
        clap_rep.reshape(n, _CLAP_DIM), W, gathered, sel.reshape(n, 1)
    )
    return out.reshape(b, s, _EMB_DIM)


# speedup vs baseline: 16.7809x; 1.0026x over previous
import functools

import jax
import jax.numpy as jnp
from jax import lax
from jax.experimental import pallas as pl
from jax.experimental.pallas import tpu as pltpu
from jax.experimental.pallas import tpu_sc as plsc

_EMB_DIM = 4096
_CLAP_DIM = 768

_NC, _NS = 2, 16
_NW = _NC * _NS
_CHUNK = 8


def _sc_gather(emb, ids2):
    n = ids2.shape[0] * ids2.shape[1]
    per_tile = n // _NW
    n_chunks = per_tile // _CHUNK
    v_even = emb.shape[0] - 1
    mesh = plsc.VectorSubcoreMesh(core_axis_name="c", subcore_axis_name="s")

    @functools.partial(
        pl.kernel,
        out_type=jax.ShapeDtypeStruct((n, _EMB_DIM), jnp.int32),
        mesh=mesh,
        scratch_types=[
            pltpu.VMEM((n_chunks, _CHUNK), jnp.int32),
            pltpu.VMEM((_CHUNK, _EMB_DIM), jnp.int32),
            pltpu.VMEM((_CHUNK, _EMB_DIM), jnp.int32),
            pltpu.VMEM((_CHUNK, _EMB_DIM), jnp.int32),
            pltpu.SemaphoreType.DMA,
            pltpu.SemaphoreType.DMA,
            pltpu.SemaphoreType.DMA,
            pltpu.SemaphoreType.DMA,
            pltpu.SemaphoreType.DMA,
            pltpu.SemaphoreType.DMA,
        ],
    )
    def gather_kernel(emb_hbm, ids_hbm, out_hbm, idx_v, buf0, buf1, buf2,
                      gs0, gs1, gs2, ws0, ws1, ws2):
        emb_i32 = emb_hbm.at[pl.ds(0, v_even)].bitcast(jnp.int32)
        wid = lax.axis_index("s") * _NC + lax.axis_index("c")
        base = wid * per_tile

        pltpu.sync_copy(ids_hbm.at[pl.ds(wid * n_chunks, n_chunks)], idx_v)

        def gather(c, buf, sem):
            pltpu.async_copy(emb_i32.at[idx_v.at[c]], buf, sem)

        def gather_wait(c, buf, sem):
            pltpu.make_async_copy(emb_i32.at[idx_v.at[c]], buf, sem).wait()

        def _out_slice(c):
            off = pl.multiple_of(base + c * _CHUNK, _CHUNK)
            return out_hbm.at[pl.ds(off, _CHUNK)]

        def wback(c, buf, sem):
            pltpu.async_copy(buf, _out_slice(c), sem)

        def wback_wait(c, buf, sem):
            pltpu.make_async_copy(buf, _out_slice(c), sem).wait()

        bufs = (buf0, buf1, buf2)
        gsems = (gs0, gs1, gs2)
        wsems = (ws0, ws1, ws2)

        n_loop = n_chunks // 3
        for k in range(3):
            gather(k, bufs[k], gsems[k])

        @pl.loop(0, n_loop)
        def _(i):
            c0 = 3 * i
            for k in range(3):
                gather_wait(c0 + k, bufs[k], gsems[k])
                wback(c0 + k, bufs[k], wsems[k])
            for k in range(3):

                @pl.when(c0 + 3 + k < n_chunks)
                def _(k=k):
                    wback_wait(c0 + k, bufs[k], wsems[k])
                    gather(c0 + 3 + k, bufs[k], gsems[k])

        for c in range(3 * n_loop, n_chunks):
            k = c % 3
            gather_wait(c, bufs[k], gsems[k])
            wback(c, bufs[k], wsems[k])
        for c in range(n_chunks - 3, n_chunks):
            k = c % 3
            wback_wait(c, bufs[k], wsems[k])

    return gather_kernel(emb, ids2)


def _tc_project_merge(clap, w, gathered, sel):
    n = clap.shape[0]
    blk = 512
    grid = (n // blk,)

    def body(clap_ref, w_ref, gath_ref, sel_ref, out_ref):
        audio = lax.dot_general(
            clap_ref[...], w_ref[...],
            (((1,), (1,)), ((), ())),
            preferred_element_type=jnp.float32,
        )
        sel = sel_ref[...]
        words = gath_ref[...]
        h = jnp.where(sel > 0,
                      lax.shift_right_logical(words, 16),
                      words & jnp.int32(0xFFFF))
        f32bits = (lax.shift_left(h & jnp.int32(0x8000), 16)
                   | lax.shift_left(h & jnp.int32(0x7FFF), 13))
        gf = lax.bitcast_convert_type(f32bits, jnp.float32) * jnp.float32(
            2.0**112)
        out_ref[...] = jnp.where(sel < 0, audio, gf)

    return pl.pallas_call(
        body,
        grid=grid,
        in_specs=[
            pl.BlockSpec((blk, _CLAP_DIM), lambda i: (i, 0)),
            pl.BlockSpec((_EMB_DIM, _CLAP_DIM), lambda i: (0, 0)),
            pl.BlockSpec((blk, _EMB_DIM), lambda i: (i, 0)),
            pl.BlockSpec((blk, 1), lambda i: (i, 0)),
        ],
        out_specs=pl.BlockSpec((blk, _EMB_DIM), lambda i: (i, 0)),
        out_shape=jax.ShapeDtypeStruct((n, _EMB_DIM), jnp.float32),
    )(clap, w, gathered, sel)


def kernel(input_ids, clap_rep, pos_id, emb, W):
    b, s = input_ids.shape
    n = b * s
    per_tile = n // _NW
    n_chunks = per_tile // _CHUNK
    a_content = emb.shape[0] - 1
    ids = input_ids.reshape(n)
    audio = pos_id.reshape(n) > 0
    word_ids = jnp.where(audio, 0, ids) >> 1
    sel = jnp.where(audio, -1, ids & 1).astype(jnp.int32)
    gathered = _sc_gather(emb, word_ids.reshape(_NW * n_chunks, _CHUNK))
    out = _tc_project_merge(
        clap_rep.reshape(n, _CLAP_DIM), W, gathered, sel.reshape(n, 1)
    )
    return out.reshape(b, s, _EMB_DIM)
